# baseline (speedup 1.0000x reference)
import functools

import jax
import jax.numpy as jnp
from jax import lax
from jax.experimental import pallas as pl
from jax.experimental.pallas import tpu as pltpu
from jax.experimental.pallas import tpu_sc as plsc

B = 16384
D = 32
NC = 2
NS = 16
NW = NC * NS
BPW = B // NW
L = 16
SR = 250000
GC = 128
NG = BPW // GC


def _mf_body(user_hbm, item_hbm, ut_hbm, it_hbm, out_hbm,
             uidx, iidx, usr, isr, urows, irows, outv, sem):
    wid = lax.axis_index("s") * NC + lax.axis_index("c")
    base = wid * BPW

    pltpu.sync_copy(user_hbm.at[wid], uidx)
    pltpu.sync_copy(item_hbm.at[wid], iidx)

    for k in range(BPW // L):
        g, o = divmod(k * L, GC)
        usr[g, pl.ds(o, L)] = uidx[pl.ds(k * L, L)] >> 2
        isr[g, pl.ds(o, L)] = iidx[pl.ds(k * L, L)] >> 2

    def take(v, p):
        return v.at[p].get(mode="promise_in_bounds")

    def chunk(g, carry):
        cu = pltpu.async_copy(ut_hbm.at[usr.at[g]], urows, sem)
        ci = pltpu.async_copy(it_hbm.at[isr.at[g]], irows, sem)
        cu.wait()
        ci.wait()

        lane = lax.iota(jnp.int32, L)
        halflo = lane < (L // 2)
        perms = []
        for k in range(4):
            s = 3 - k
            hn = 1 << s
            p1 = (((lane & 7) >> s) << (s + 1)) | (lane & (hn - 1))
            perms.append((p1, p1 + hn))

        for grp in range(GC // L):
            r0 = g * GC + grp * L
            uo = (uidx[pl.ds(r0, L)] & 3) * D
            io = (iidx[pl.ds(r0, L)] & 3) * D
            vs = []
            for k in range(L):
                j = grp * L + k
                ua = uo[k]
                ia = io[k]
                vs.append(
                    urows[j, pl.ds(ua, L)] * irows[j, pl.ds(ia, L)]
                    + urows[j, pl.ds(ua + L, L)] * irows[j, pl.ds(ia + L, L)])
            for k in range(4):
                p1, p2 = perms[k]
                nxt = []
                for j in range(len(vs) // 2):
                    a, b = vs[2 * j], vs[2 * j + 1]
                    fa = take(a, p1) + take(a, p2)
                    fb = take(b, p1) + take(b, p2)
                    nxt.append(jnp.where(halflo, fa, fb))
                vs = nxt
            outv[pl.ds(r0, L)] = vs[0]
        return carry

    lax.fori_loop(0, NG, chunk, 0)

    pltpu.sync_copy(outv, out_hbm.at[pl.ds(base, BPW)])


@functools.partial(
    pl.kernel,
    out_type=jax.ShapeDtypeStruct((B,), jnp.float32),
    mesh=plsc.VectorSubcoreMesh(core_axis_name="c", subcore_axis_name="s"),
    scratch_types=[
        pltpu.VMEM((BPW,), jnp.int32),
        pltpu.VMEM((BPW,), jnp.int32),
        pltpu.VMEM((NG, GC), jnp.int32),
        pltpu.VMEM((NG, GC), jnp.int32),
        pltpu.VMEM((GC, 4 * D), jnp.float32),
        pltpu.VMEM((GC, 4 * D), jnp.float32),
        pltpu.VMEM((BPW,), jnp.float32),
        pltpu.SemaphoreType.DMA,
    ],
)
def _mf_kernel(user_hbm, item_hbm, ut_hbm, it_hbm, out_hbm,
               uidx, iidx, usr, isr, urows, irows, outv, sem):
    _mf_body(user_hbm, item_hbm, ut_hbm, it_hbm, out_hbm,
             uidx, iidx, usr, isr, urows, irows, outv, sem)


def kernel(user, item, user_table, item_table):
    u = user.astype(jnp.int32).reshape(NW, BPW)
    it = item.astype(jnp.int32).reshape(NW, BPW)
    ut2 = user_table.reshape(SR, 4 * D)
    it2 = item_table.reshape(SR, 4 * D)
    return _mf_kernel(u, it, ut2, it2)


# speedup vs baseline: 2.3116x; 2.3116x over previous
import functools

import jax
import jax.numpy as jnp
from jax import lax
from jax.experimental import pallas as pl
from jax.experimental.pallas import tpu as pltpu
from jax.experimental.pallas import tpu_sc as plsc

B = 16384
D = 32
NC = 2
NS = 16
NW = NC * NS
BPW = B // NW
L = 16
VPT = 125000
NCH = BPW // L


def _mf_body(user_hbm, item_hbm, ut_hbm, it_hbm, out_hbm,
             uidx, iidx, utiles, itiles, outv, sem0, sem1):
    wid = lax.axis_index("s") * NC + lax.axis_index("c")
    base = wid * BPW

    pltpu.sync_copy(user_hbm.at[wid], uidx)
    pltpu.sync_copy(item_hbm.at[wid], iidx)

    sems = [sem0, sem1]

    def fire(g, buf):
        sem = sems[buf]
        uv = uidx[pl.ds(g * L, L)] >> 3
        iv = iidx[pl.ds(g * L, L)] >> 3
        for k in range(L):
            pltpu.async_copy(ut_hbm.at[uv[k]], utiles.at[buf, k], sem)
            pltpu.async_copy(it_hbm.at[iv[k]], itiles.at[buf, k], sem)

    def drain(buf):
        sem = sems[buf]
        pltpu.make_async_copy(ut_hbm.at[pl.ds(0, L)], utiles.at[buf], sem).wait()
        pltpu.make_async_copy(it_hbm.at[pl.ds(0, L)], itiles.at[buf], sem).wait()

    def take(v, p):
        return v.at[p].get(mode="promise_in_bounds")

    def compute(g, buf):
        lane = lax.iota(jnp.int32, L)
        halflo = lane < (L // 2)
        perms = []
        for k in range(4):
            s = 3 - k
            hn = 1 << s
            p1 = (((lane & 7) >> s) << (s + 1)) | (lane & (hn - 1))
            perms.append((p1, p1 + hn))
        uv = uidx[pl.ds(g * L, L)] & 7
        iv = iidx[pl.ds(g * L, L)] & 7
        vs = []
        for k in range(L):
            ur = uv[k]
            ir = iv[k]
            vs.append(
                utiles[buf, k, ur, pl.ds(0, L)]
                * itiles[buf, k, ir, pl.ds(0, L)]
                + utiles[buf, k, ur, pl.ds(L, L)]
                * itiles[buf, k, ir, pl.ds(L, L)])
        for k in range(4):
            p1, p2 = perms[k]
            nxt = []
            for j in range(len(vs) // 2):
                a, b = vs[2 * j], vs[2 * j + 1]
                fa = take(a, p1) + take(a, p2)
                fb = take(b, p1) + take(b, p2)
                nxt.append(jnp.where(halflo, fa, fb))
            vs = nxt
        outv[pl.ds(g * L, L)] = vs[0]

    fire(0, 0)

    def step(g, carry):
        even = lax.rem(g, 2) == 0

        @pl.when(even)
        def _():
            @pl.when(g + 1 < NCH)
            def _():
                fire(g + 1, 1)
            drain(0)
            compute(g, 0)

        @pl.when(jnp.logical_not(even))
        def _():
            @pl.when(g + 1 < NCH)
            def _():
                fire(g + 1, 0)
            drain(1)
            compute(g, 1)
        return carry

    lax.fori_loop(0, NCH, step, 0)

    pltpu.sync_copy(outv, out_hbm.at[pl.ds(base, BPW)])


@functools.partial(
    pl.kernel,
    out_type=jax.ShapeDtypeStruct((B,), jnp.float32),
    mesh=plsc.VectorSubcoreMesh(core_axis_name="c", subcore_axis_name="s"),
    scratch_types=[
        pltpu.VMEM((BPW,), jnp.int32),
        pltpu.VMEM((BPW,), jnp.int32),
        pltpu.VMEM((2, L, 8, D), jnp.float32),
        pltpu.VMEM((2, L, 8, D), jnp.float32),
        pltpu.VMEM((BPW,), jnp.float32),
        pltpu.SemaphoreType.DMA,
        pltpu.SemaphoreType.DMA,
    ],
)
def _mf_kernel(user_hbm, item_hbm, ut_hbm, it_hbm, out_hbm,
               uidx, iidx, utiles, itiles, outv, sem0, sem1):
    _mf_body(user_hbm, item_hbm, ut_hbm, it_hbm, out_hbm,
             uidx, iidx, utiles, itiles, outv, sem0, sem1)


def kernel(user, item, user_table, item_table):
    u = user.astype(jnp.int32).reshape(NW, BPW)
    it = item.astype(jnp.int32).reshape(NW, BPW)
    ut3 = user_table.reshape(VPT, 8, D)
    it3 = item_table.reshape(VPT, 8, D)
    return _mf_kernel(u, it, ut3, it3)
